# baseline (speedup 1.0000x reference)
import functools

import jax
import jax.numpy as jnp
from jax import lax
from jax.experimental import pallas as pl
from jax.experimental.pallas import tpu as pltpu
from jax.experimental.pallas import tpu_sc as plsc

S = 2048
D = 768
OUT = 768
E = 8
K = 2
TB = 256
NPAIR = S * K
NBUF = NPAIR + E * TB
NBLK = NBUF // TB
NW = 32
CB = 256


def _gate_body(x_ref, gw_ref, gb_ref, d0_ref, d1_ref, p0_ref, p1_ref,
               be_ref):
    x = x_ref[...]
    gw = gw_ref[...]
    logits = lax.dot_general(gw, x, (((1,), (1,)), ((), ())),
                             preferred_element_type=jnp.float32)
    logits = logits + gb_ref[...]
    m = jnp.max(logits, axis=0, keepdims=True)
    ex = jnp.exp(logits - m)
    p = ex / jnp.sum(ex, axis=0, keepdims=True)
    ii = lax.broadcasted_iota(jnp.int32, (E, S), 0)
    m1 = jnp.max(p, axis=0, keepdims=True)
    i1 = jnp.min(jnp.where(p == m1, ii, E), axis=0, keepdims=True)
    pm = jnp.where(ii == i1, -1.0, p)
    m2 = jnp.max(pm, axis=0, keepdims=True)
    i2 = jnp.min(jnp.where(pm == m2, ii, E), axis=0, keepdims=True)
    p0_ref[...] = jnp.sum(jnp.where(ii == 0, p, 0.0), axis=0)
    p1_ref[...] = jnp.sum(jnp.where(ii == 1, p, 0.0), axis=0)

    oh = (ii == i1).astype(jnp.float32) + (ii == i2).astype(jnp.float32)
    ri = lax.broadcasted_iota(jnp.int32, (CB, CB), 0)
    ci = lax.broadcasted_iota(jnp.int32, (CB, CB), 1)
    ut = (ri <= ci).astype(jnp.float32)
    blocks = []
    carry = jnp.zeros((E, 1), jnp.float32)
    for c in range(S // CB):
        blk = oh[:, c * CB:(c + 1) * CB]
        cum = lax.dot_general(blk, ut, (((1,), (0,)), ((), ())),
                              preferred_element_type=jnp.float32) + carry
        blocks.append(cum)
        carry = cum[:, CB - 1:CB]
    ct = jnp.concatenate(blocks, axis=1)

    counts = ct[:, S - 1:S]
    pc = jnp.floor((counts + (TB - 1)) * (1.0 / TB)) * TB
    ii8 = lax.broadcasted_iota(jnp.int32, (E, E), 0)
    jj8 = lax.broadcasted_iota(jnp.int32, (E, E), 1)
    cummat = (jj8 <= ii8).astype(jnp.float32)
    ends = lax.dot_general(cummat, pc, (((1,), (0,)), ((), ())),
                           preferred_element_type=jnp.float32)
    starts = ends - pc

    slot = ct + starts - 1.0
    d0 = jnp.sum(jnp.where(ii == i1, slot, 0.0), axis=0)
    d1 = jnp.sum(jnp.where(ii == i2, slot, 0.0), axis=0)
    d0_ref[...] = d0.astype(jnp.int32)
    d1_ref[...] = d1.astype(jnp.int32)

    bi = lax.broadcasted_iota(jnp.int32, (E, 32), 1).astype(jnp.float32) * float(TB)
    be = jnp.sum((ends <= bi).astype(jnp.int32), axis=0)
    be = jnp.minimum(be, E - 1)
    jj32 = lax.broadcasted_iota(jnp.int32, (E, 32), 1)
    ii32 = lax.broadcasted_iota(jnp.int32, (E, 32), 0)
    total = jnp.sum(jnp.where((jj32 == 31) & (ii32 == E - 1),
                              ends * (1.0 / TB), 0.0), axis=0).astype(jnp.int32)
    be_ref[...] = jnp.where(jnp.arange(32) == 31, total, be)


def _gate(x2d, gate_w, gate_b):
    return pl.pallas_call(
        _gate_body,
        out_shape=(
            jax.ShapeDtypeStruct((S,), jnp.int32),
            jax.ShapeDtypeStruct((S,), jnp.int32),
            jax.ShapeDtypeStruct((S,), jnp.float32),
            jax.ShapeDtypeStruct((S,), jnp.float32),
            jax.ShapeDtypeStruct((32,), jnp.int32),
        ),
    )(x2d, gate_w, gate_b.reshape(E, 1))


_X_PER_W = S // NW


@functools.cache
def _sc_mesh():
    return plsc.VectorSubcoreMesh(core_axis_name="c", subcore_axis_name="s")


@functools.cache
def _sc_scatter_fn():
    @functools.partial(
        pl.kernel,
        out_type=jax.ShapeDtypeStruct((NBUF, D), jnp.float32),
        mesh=_sc_mesh(),
        scratch_types=[
            pltpu.VMEM((_X_PER_W, D), jnp.float32),
            pltpu.VMEM((_X_PER_W,), jnp.int32),
            pltpu.VMEM((_X_PER_W,), jnp.int32),
            pltpu.SemaphoreType.DMA,
            pltpu.SemaphoreType.DMA,
        ],
    )
    def scatter(x_hbm, d0_hbm, d1_hbm, out_hbm, xrows_v, i0_v, i1_v,
                sem0, sem1):
        wid = lax.axis_index("s") * 2 + lax.axis_index("c")
        base = wid * _X_PER_W
        pltpu.sync_copy(x_hbm.at[pl.ds(base, _X_PER_W)], xrows_v)
        pltpu.sync_copy(d0_hbm.at[pl.ds(base, _X_PER_W)], i0_v)
        pltpu.sync_copy(d1_hbm.at[pl.ds(base, _X_PER_W)], i1_v)
        c0 = pltpu.async_copy(xrows_v, out_hbm.at[i0_v], sem0)
        c1 = pltpu.async_copy(xrows_v, out_hbm.at[i1_v], sem1)
        c0.wait()
        c1.wait()

    return scatter


def _sc_scatter(x2d, d0, d1):
    return _sc_scatter_fn()(x2d, d0, d1)


CH = OUT // 3


def _mlp_body(be_ref, xg_ref, w1_ref, b1_ref, w2_ref, b2_ref,
              w3_ref, b3_ref, out_ref, h1_ref, h2_ref):
    i = pl.program_id(0)
    j = pl.program_id(1)

    @pl.when(i < be_ref[31])
    def _():
        l = j // 3
        c = j % 3

        @pl.when(l == 0)
        def _():
            h = lax.dot_general(xg_ref[...], w1_ref[0], (((1,), (1,)), ((), ())),
                                preferred_element_type=jnp.float32)
            h1_ref[c] = jnp.maximum(h + b1_ref[0], 0.0)

        @pl.when(l == 1)
        def _():
            h1 = jnp.concatenate([h1_ref[0], h1_ref[1], h1_ref[2]], axis=1)
            h = lax.dot_general(h1, w2_ref[0], (((1,), (1,)), ((), ())),
                                preferred_element_type=jnp.float32)
            h2_ref[c] = jnp.maximum(h + b2_ref[0], 0.0)

        @pl.when(l == 2)
        def _():
            h2 = jnp.concatenate([h2_ref[0], h2_ref[1], h2_ref[2]], axis=1)
            h = lax.dot_general(h2, w3_ref[0], (((1,), (1,)), ((), ())),
                                preferred_element_type=jnp.float32)
            out_ref[...] = h + b3_ref[0]


def _c1(j):
    return jnp.minimum(j, 2)


def _c2(j):
    return jnp.clip(j - 3, 0, 2)


def _c3(j):
    return jnp.clip(j - 6, 0, 2)


def _mlp(xg, benv, fc1_w, fc1_b, fc2_w, fc2_b, fc3_w, fc3_b):
    grid_spec = pltpu.PrefetchScalarGridSpec(
        num_scalar_prefetch=1,
        grid=(NBLK, 9),
        in_specs=[
            pl.BlockSpec((TB, D), lambda i, j, be: (i, 0)),
            pl.BlockSpec((1, CH, D), lambda i, j, be: (be[i], _c1(j), 0)),
            pl.BlockSpec((1, 1, CH), lambda i, j, be: (be[i], 0, _c1(j))),
            pl.BlockSpec((1, CH, OUT), lambda i, j, be: (be[i], _c2(j), 0)),
            pl.BlockSpec((1, 1, CH), lambda i, j, be: (be[i], 0, _c2(j))),
            pl.BlockSpec((1, CH, OUT), lambda i, j, be: (be[i], _c3(j), 0)),
            pl.BlockSpec((1, 1, CH), lambda i, j, be: (be[i], 0, _c3(j))),
        ],
        out_specs=pl.BlockSpec((TB, CH), lambda i, j, be: (i, _c3(j))),
        scratch_shapes=[
            pltpu.VMEM((3, TB, CH), jnp.float32),
            pltpu.VMEM((3, TB, CH), jnp.float32),
        ],
    )
    return pl.pallas_call(
        _mlp_body,
        grid_spec=grid_spec,
        out_shape=jax.ShapeDtypeStruct((NBUF, OUT), jnp.float32),
    )(benv, xg, fc1_w, fc1_b.reshape(E, 1, OUT),
      fc2_w, fc2_b.reshape(E, 1, OUT), fc3_w, fc3_b.reshape(E, 1, OUT))


_C_PER_W = S // NW
_NL = 16


def _lane_bcast(v, l):
    idx = jnp.full((_NL,), l, jnp.int32)
    return lax.gather(
        v, idx[:, None],
        lax.GatherDimensionNumbers(offset_dims=(), collapsed_slice_dims=(0,),
                                   start_index_map=(0,)),
        (1,), mode=lax.GatherScatterMode.PROMISE_IN_BOUNDS)


@functools.cache
def _sc_combine_fn():
    @functools.partial(
        pl.kernel,
        out_type=jax.ShapeDtypeStruct((S, OUT), jnp.float32),
        mesh=_sc_mesh(),
        scratch_types=[
            pltpu.VMEM((_C_PER_W,), jnp.int32),
            pltpu.VMEM((_C_PER_W,), jnp.int32),
            pltpu.VMEM((_C_PER_W,), jnp.float32),
            pltpu.VMEM((_C_PER_W,), jnp.float32),
            pltpu.VMEM((_C_PER_W, OUT), jnp.float32),
            pltpu.VMEM((_C_PER_W, OUT), jnp.float32),
            pltpu.SemaphoreType.DMA,
            pltpu.SemaphoreType.DMA,
        ],
    )
    def combine(ybuf_hbm, d0_hbm, d1_hbm, p0_hbm, p1_hbm, out_hbm,
                i0_v, i1_v, p0_v, p1_v, r0_v, r1_v, sem0, sem1):
        wid = lax.axis_index("s") * 2 + lax.axis_index("c")
        base = wid * _C_PER_W
        pltpu.sync_copy(d0_hbm.at[pl.ds(base, _C_PER_W)], i0_v)
        pltpu.sync_copy(d1_hbm.at[pl.ds(base, _C_PER_W)], i1_v)
        c0 = pltpu.async_copy(ybuf_hbm.at[i0_v], r0_v, sem0)
        c1 = pltpu.async_copy(ybuf_hbm.at[i1_v], r1_v, sem1)
        pltpu.sync_copy(p0_hbm.at[pl.ds(base, _C_PER_W)], p0_v)
        pltpu.sync_copy(p1_hbm.at[pl.ds(base, _C_PER_W)], p1_v)
        c0.wait()
        c1.wait()

        @plsc.parallel_loop(0, _C_PER_W, 1, unroll=2)
        def _(t):
            cbase = (t // _NL) * _NL
            lane = t - cbase
            b0 = _lane_bcast(p0_v[pl.ds(cbase, _NL)], lane)
            b1 = _lane_bcast(p1_v[pl.ds(cbase, _NL)], lane)
            for j in range(OUT // _NL):
                sl = pl.ds(j * _NL, _NL)
                r0_v[t, sl] = r0_v[t, sl] * b0 + r1_v[t, sl] * b1
        pltpu.sync_copy(r0_v, out_hbm.at[pl.ds(base, _C_PER_W)])

    return combine


def _sc_combine(ybuf, d0, d1, p0, p1):
    return _sc_combine_fn()(ybuf, d0, d1, p0, p1)


def kernel(x, gate_w, gate_b, fc1_w, fc1_b, fc2_w, fc2_b, fc3_w, fc3_b):
    x2d = x.reshape(S, D)
    d0, d1, p0, p1, benv = _gate(x2d, gate_w, gate_b)
    xg = _sc_scatter(x2d, d0, d1)
    ybuf = _mlp(xg, benv, fc1_w, fc1_b, fc2_w, fc2_b, fc3_w, fc3_b)
    out = _sc_combine(ybuf, d0, d1, p0, p1)
    return out.reshape(1, S, OUT)


# speedup vs baseline: 2.5328x; 2.5328x over previous
import functools

import jax
import jax.numpy as jnp
from jax import lax
from jax.experimental import pallas as pl
from jax.experimental.pallas import tpu as pltpu
from jax.experimental.pallas import tpu_sc as plsc

S = 2048
D = 768
OUT = 768
E = 8
K = 2
TB = 256
NPAIR = S * K
NBUF = NPAIR + E * TB
NBLK = NBUF // TB
NW = 32
CB = 256


def _gate_body(x_ref, gw_ref, gb_ref, d0_ref, d1_ref, p0_ref, p1_ref,
               be_ref, sl_ref, eo_ref):
    x = x_ref[...]
    gw = gw_ref[...]
    logits = lax.dot_general(gw, x, (((1,), (1,)), ((), ())),
                             preferred_element_type=jnp.float32)
    logits = logits + gb_ref[...]
    m = jnp.max(logits, axis=0, keepdims=True)
    ex = jnp.exp(logits - m)
    p = ex / jnp.sum(ex, axis=0, keepdims=True)
    ii = lax.broadcasted_iota(jnp.int32, (E, S), 0)
    m1 = jnp.max(p, axis=0, keepdims=True)
    i1 = jnp.min(jnp.where(p == m1, ii, E), axis=0, keepdims=True)
    pm = jnp.where(ii == i1, -1.0, p)
    m2 = jnp.max(pm, axis=0, keepdims=True)
    i2 = jnp.min(jnp.where(pm == m2, ii, E), axis=0, keepdims=True)
    p0_ref[...] = jnp.sum(jnp.where(ii == 0, p, 0.0), axis=0)
    p1_ref[...] = jnp.sum(jnp.where(ii == 1, p, 0.0), axis=0)

    oh = (ii == i1).astype(jnp.float32) + (ii == i2).astype(jnp.float32)
    ri = lax.broadcasted_iota(jnp.int32, (CB, CB), 0)
    ci = lax.broadcasted_iota(jnp.int32, (CB, CB), 1)
    ut = (ri <= ci).astype(jnp.float32)
    blocks = []
    carry = jnp.zeros((E, 1), jnp.float32)
    for c in range(S // CB):
        blk = oh[:, c * CB:(c + 1) * CB]
        cum = lax.dot_general(blk, ut, (((1,), (0,)), ((), ())),
                              preferred_element_type=jnp.float32) + carry
        blocks.append(cum)
        carry = cum[:, CB - 1:CB]
    ct = jnp.concatenate(blocks, axis=1)

    counts = ct[:, S - 1:S]
    pc = jnp.floor((counts + (TB - 1)) * (1.0 / TB)) * TB
    ii8 = lax.broadcasted_iota(jnp.int32, (E, E), 0)
    jj8 = lax.broadcasted_iota(jnp.int32, (E, E), 1)
    cummat = (jj8 <= ii8).astype(jnp.float32)
    ends = lax.dot_general(cummat, pc, (((1,), (0,)), ((), ())),
                           preferred_element_type=jnp.float32)
    starts = ends - pc

    slot = ct + starts - 1.0
    d0 = jnp.sum(jnp.where(ii == i1, slot, 0.0), axis=0)
    d1 = jnp.sum(jnp.where(ii == i2, slot, 0.0), axis=0)
    d0_ref[...] = d0.astype(jnp.int32)
    d1_ref[...] = d1.astype(jnp.int32)

    bi = lax.broadcasted_iota(jnp.int32, (E, 32), 1).astype(jnp.float32) * float(TB)
    be = jnp.sum((ends <= bi).astype(jnp.int32), axis=0)
    be = jnp.minimum(be, E - 1)
    jj32 = lax.broadcasted_iota(jnp.int32, (E, 32), 1)
    ii32 = lax.broadcasted_iota(jnp.int32, (E, 32), 0)
    total = jnp.sum(jnp.where((jj32 == 31) & (ii32 == E - 1),
                              ends * (1.0 / TB), 0.0), axis=0).astype(jnp.int32)
    be_ref[...] = jnp.where(jnp.arange(32) == 31, total, be)

    live = (pc > 0.0)
    ii8f = ii8.astype(jnp.float32)
    jj8f = jj8.astype(jnp.float32)
    slt = (jj8f < ii8f).astype(jnp.float32)
    rk = lax.dot_general(slt, jnp.where(live, 1.0, 0.0),
                         (((1,), (0,)), ((), ())),
                         preferred_element_type=jnp.float32)
    live_b = jnp.broadcast_to(live, (E, 32))
    rk_b = jnp.broadcast_to(rk.astype(jnp.int32), (E, 32))
    eo_raw = jnp.sum(jnp.where(live_b & (rk_b == jj32), ii32, 0), axis=0)
    nexp = jnp.sum(jnp.where(live, 1, 0).astype(jnp.int32))
    ar32 = jnp.arange(32)
    eo = jnp.where(ar32 < nexp, eo_raw, E - 1)
    eo_ref[...] = jnp.where(ar32 == 8, nexp, eo)
    bsl = jnp.sum(jnp.where(live_b & (jnp.broadcast_to(ends, (E, 32)) <= bi),
                            1, 0), axis=0)
    fbv = jnp.sum(jnp.where(live_b & (jnp.broadcast_to(starts, (E, 32)) == bi),
                            1, 0), axis=0)
    sl_ref[...] = bsl * 2 + fbv


def _gate(x2d, gate_w, gate_b):
    return pl.pallas_call(
        _gate_body,
        out_shape=(
            jax.ShapeDtypeStruct((S,), jnp.int32),
            jax.ShapeDtypeStruct((S,), jnp.int32),
            jax.ShapeDtypeStruct((S,), jnp.float32),
            jax.ShapeDtypeStruct((S,), jnp.float32),
            jax.ShapeDtypeStruct((32,), jnp.int32),
            jax.ShapeDtypeStruct((32,), jnp.int32),
            jax.ShapeDtypeStruct((32,), jnp.int32),
        ),
    )(x2d, gate_w, gate_b.reshape(E, 1))


_X_PER_W = S // NW


@functools.cache
def _sc_mesh():
    return plsc.VectorSubcoreMesh(core_axis_name="c", subcore_axis_name="s")


@functools.cache
def _sc_scatter_fn():
    @functools.partial(
        pl.kernel,
        out_type=jax.ShapeDtypeStruct((NBUF, D), jnp.float32),
        mesh=_sc_mesh(),
        scratch_types=[
            pltpu.VMEM((_X_PER_W, D), jnp.float32),
            pltpu.VMEM((_X_PER_W,), jnp.int32),
            pltpu.VMEM((_X_PER_W,), jnp.int32),
            pltpu.SemaphoreType.DMA,
            pltpu.SemaphoreType.DMA,
        ],
    )
    def scatter(x_hbm, d0_hbm, d1_hbm, out_hbm, xrows_v, i0_v, i1_v,
                sem0, sem1):
        wid = lax.axis_index("s") * 2 + lax.axis_index("c")
        base = wid * _X_PER_W
        pltpu.sync_copy(x_hbm.at[pl.ds(base, _X_PER_W)], xrows_v)
        pltpu.sync_copy(d0_hbm.at[pl.ds(base, _X_PER_W)], i0_v)
        pltpu.sync_copy(d1_hbm.at[pl.ds(base, _X_PER_W)], i1_v)
        c0 = pltpu.async_copy(xrows_v, out_hbm.at[i0_v], sem0)
        c1 = pltpu.async_copy(xrows_v, out_hbm.at[i1_v], sem1)
        c0.wait()
        c1.wait()

    return scatter


def _sc_scatter(x2d, d0, d1):
    return _sc_scatter_fn()(x2d, d0, d1)


def _mlp_body(be_ref, sl_ref, eo_ref, xg_ref, b1_ref, b2_ref, b3_ref,
              w1_hbm, w2_hbm, w3_hbm, out_ref,
              w1b, w2b, w3b, sem1, sem2, sem3):
    i = pl.program_id(0)
    nv = be_ref[31]
    nexp = eo_ref[8]
    svf = sl_ref[i]
    s = svf // 2
    fb = svf - 2 * s

    def fetch(slot):
        @pl.when(slot < nexp)
        def _():
            e = eo_ref[jnp.minimum(slot, 7)]
            k = lax.rem(slot, 3)
            pltpu.make_async_copy(w1_hbm.at[e], w1b.at[k], sem1.at[k]).start()
            pltpu.make_async_copy(w2_hbm.at[e], w2b.at[k], sem2.at[k]).start()
            pltpu.make_async_copy(w3_hbm.at[e], w3b.at[k], sem3.at[k]).start()

    @pl.when(i == 0)
    def _():
        fetch(jnp.int32(0))
        fetch(jnp.int32(1))

    @pl.when((i < nv) & (fb == 1))
    def _():
        fetch(s + 2)
        e = eo_ref[jnp.minimum(s, 7)]
        k = lax.rem(s, 3)
        pltpu.make_async_copy(w1_hbm.at[e], w1b.at[k], sem1.at[k]).wait()
        pltpu.make_async_copy(w2_hbm.at[e], w2b.at[k], sem2.at[k]).wait()
        pltpu.make_async_copy(w3_hbm.at[e], w3b.at[k], sem3.at[k]).wait()

    @pl.when(i < nv)
    def _():
        k = lax.rem(s, 3)
        xb = xg_ref[...]
        h = lax.dot_general(xb, w1b[k], (((1,), (1,)), ((), ())),
                            preferred_element_type=jnp.float32)
        h = jnp.maximum(h + b1_ref[0], 0.0)
        h = lax.dot_general(h, w2b[k], (((1,), (1,)), ((), ())),
                            preferred_element_type=jnp.float32)
        h = jnp.maximum(h + b2_ref[0], 0.0)
        h = lax.dot_general(h, w3b[k], (((1,), (1,)), ((), ())),
                            preferred_element_type=jnp.float32)
        out_ref[...] = h + b3_ref[0]


def _mlp(xg, benv, sl, eo, fc1_w, fc1_b, fc2_w, fc2_b, fc3_w, fc3_b):
    grid_spec = pltpu.PrefetchScalarGridSpec(
        num_scalar_prefetch=3,
        grid=(NBLK,),
        in_specs=[
            pl.BlockSpec((TB, D), lambda i, be, sl, eo: (i, 0)),
            pl.BlockSpec((1, 1, OUT), lambda i, be, sl, eo: (be[i], 0, 0)),
            pl.BlockSpec((1, 1, OUT), lambda i, be, sl, eo: (be[i], 0, 0)),
            pl.BlockSpec((1, 1, OUT), lambda i, be, sl, eo: (be[i], 0, 0)),
            pl.BlockSpec(memory_space=pl.ANY),
            pl.BlockSpec(memory_space=pl.ANY),
            pl.BlockSpec(memory_space=pl.ANY),
        ],
        out_specs=pl.BlockSpec((TB, OUT), lambda i, be, sl, eo: (i, 0)),
        scratch_shapes=[
            pltpu.VMEM((3, OUT, D), jnp.float32),
            pltpu.VMEM((3, OUT, OUT), jnp.float32),
            pltpu.VMEM((3, OUT, OUT), jnp.float32),
            pltpu.SemaphoreType.DMA((3,)),
            pltpu.SemaphoreType.DMA((3,)),
            pltpu.SemaphoreType.DMA((3,)),
        ],
    )
    return pl.pallas_call(
        _mlp_body,
        grid_spec=grid_spec,
        out_shape=jax.ShapeDtypeStruct((NBUF, OUT), jnp.float32),
    )(benv, sl, eo, xg, fc1_b.reshape(E, 1, OUT), fc2_b.reshape(E, 1, OUT),
      fc3_b.reshape(E, 1, OUT), fc1_w, fc2_w, fc3_w)


_C_PER_W = S // NW
_NL = 16


def _lane_bcast(v, l):
    idx = jnp.full((_NL,), l, jnp.int32)
    return lax.gather(
        v, idx[:, None],
        lax.GatherDimensionNumbers(offset_dims=(), collapsed_slice_dims=(0,),
                                   start_index_map=(0,)),
        (1,), mode=lax.GatherScatterMode.PROMISE_IN_BOUNDS)


@functools.cache
def _sc_combine_fn():
    @functools.partial(
        pl.kernel,
        out_type=jax.ShapeDtypeStruct((S, OUT), jnp.float32),
        mesh=_sc_mesh(),
        scratch_types=[
            pltpu.VMEM((_C_PER_W,), jnp.int32),
            pltpu.VMEM((_C_PER_W,), jnp.int32),
            pltpu.VMEM((_C_PER_W,), jnp.float32),
            pltpu.VMEM((_C_PER_W,), jnp.float32),
            pltpu.VMEM((_C_PER_W, OUT), jnp.float32),
            pltpu.VMEM((_C_PER_W, OUT), jnp.float32),
            pltpu.SemaphoreType.DMA,
            pltpu.SemaphoreType.DMA,
        ],
    )
    def combine(ybuf_hbm, d0_hbm, d1_hbm, p0_hbm, p1_hbm, out_hbm,
                i0_v, i1_v, p0_v, p1_v, r0_v, r1_v, sem0, sem1):
        wid = lax.axis_index("s") * 2 + lax.axis_index("c")
        base = wid * _C_PER_W
        pltpu.sync_copy(d0_hbm.at[pl.ds(base, _C_PER_W)], i0_v)
        pltpu.sync_copy(d1_hbm.at[pl.ds(base, _C_PER_W)], i1_v)
        c0 = pltpu.async_copy(ybuf_hbm.at[i0_v], r0_v, sem0)
        c1 = pltpu.async_copy(ybuf_hbm.at[i1_v], r1_v, sem1)
        pltpu.sync_copy(p0_hbm.at[pl.ds(base, _C_PER_W)], p0_v)
        pltpu.sync_copy(p1_hbm.at[pl.ds(base, _C_PER_W)], p1_v)
        c0.wait()
        c1.wait()

        @plsc.parallel_loop(0, _C_PER_W, 1, unroll=2)
        def _(t):
            cbase = (t // _NL) * _NL
            lane = t - cbase
            b0 = _lane_bcast(p0_v[pl.ds(cbase, _NL)], lane)
            b1 = _lane_bcast(p1_v[pl.ds(cbase, _NL)], lane)
            for j in range(OUT // _NL):
                sl = pl.ds(j * _NL, _NL)
                r0_v[t, sl] = r0_v[t, sl] * b0 + r1_v[t, sl] * b1
        pltpu.sync_copy(r0_v, out_hbm.at[pl.ds(base, _C_PER_W)])

    return combine


def _sc_combine(ybuf, d0, d1, p0, p1):
    return _sc_combine_fn()(ybuf, d0, d1, p0, p1)


def kernel(x, gate_w, gate_b, fc1_w, fc1_b, fc2_w, fc2_b, fc3_w, fc3_b):
    x2d = x.reshape(S, D)
    d0, d1, p0, p1, benv, sl, eo = _gate(x2d, gate_w, gate_b)
    xg = _sc_scatter(x2d, d0, d1)
    ybuf = _mlp(xg, benv, sl, eo, fc1_w, fc1_b, fc2_w, fc2_b, fc3_w, fc3_b)
    out = _sc_combine(ybuf, d0, d1, p0, p1)
    return out.reshape(1, S, OUT)
